# Initial kernel scaffold; baseline (speedup 1.0000x reference)
import jax
import jax.numpy as jnp
from jax.experimental import pallas as pl


def kernel(node_feature, edge_index, edge_feature, W, b):
    raise NotImplementedError("write your pallas kernel here")


# speedup vs baseline: 3.0034x; 3.0034x over previous
import functools

import jax
import jax.numpy as jnp
from jax import lax
from jax.experimental import pallas as pl
from jax.experimental.pallas import tpu as pltpu
from jax.experimental.pallas import tpu_sc as plsc

N_NODES = 10000
N_PAD = 10240
E_EDGES = 320000
D_FEAT = 128
D_HALF = D_FEAT // 2
D_EDGE = 16
D_OUT = 128

NC = 2
NS = 16
CHUNK = 80
EDGES_PER_SUBCORE = E_EDGES // NS
CHUNKS_PER_TILE = EDGES_PER_SUBCORE // CHUNK
EF_SPLIT = CHUNKS_PER_TILE // 2
ROWS_PER_TILE = N_PAD // NS

_MESH = plsc.VectorSubcoreMesh(core_axis_name="c", subcore_axis_name="s")


@functools.partial(
    pl.kernel,
    out_type=(
        jax.ShapeDtypeStruct((NC * N_PAD, D_HALF), jnp.float32),
        jax.ShapeDtypeStruct((NC * N_PAD, D_EDGE), jnp.float32),
    ),
    mesh=_MESH,
    compiler_params=pltpu.CompilerParams(use_tc_tiling_on_sc=False),
    scratch_types=[
        pltpu.VMEM((CHUNK,), jnp.int32),
        pltpu.VMEM((CHUNK,), jnp.int32),
        pltpu.VMEM((CHUNK, D_HALF), jnp.float32),
        pltpu.VMEM((CHUNK, D_EDGE), jnp.float32),
        pltpu.VMEM((ROWS_PER_TILE, D_EDGE), jnp.float32),
        pltpu.VMEM_SHARED((N_PAD, D_HALF), jnp.float32),
        pltpu.VMEM_SHARED((N_PAD, D_EDGE), jnp.float32),
        pltpu.SemaphoreType.DMA,
    ],
)
def _sc_aggregate(src_hbm, dst_hbm, xlo_hbm, xhi_hbm, ef_hbm, zx_hbm, ze_hbm,
                  outx_hbm, oute_hbm,
                  src_v, dst_v, rows_v, ef_v, eb_v, aggx_s, agge_s, sem):
    c = lax.axis_index("c")
    s = lax.axis_index("s")

    r0 = s * ROWS_PER_TILE
    pltpu.sync_copy(zx_hbm.at[pl.ds(0, CHUNK)], rows_v)
    pltpu.sync_copy(ze_hbm.at[pl.ds(0, ROWS_PER_TILE)], eb_v)
    for k in range(ROWS_PER_TILE // CHUNK):
        pltpu.sync_copy(rows_v, aggx_s.at[pl.ds(r0 + k * CHUNK, CHUNK)])
    pltpu.sync_copy(eb_v, agge_s.at[pl.ds(r0, ROWS_PER_TILE)])
    plsc.subcore_barrier()

    sub_base = s * EDGES_PER_SUBCORE

    def body(g, carry):
        base = sub_base + g * CHUNK
        pltpu.sync_copy(src_hbm.at[pl.ds(base, CHUNK)], src_v)
        pltpu.sync_copy(dst_hbm.at[pl.ds(base, CHUNK)], dst_v)
        @pl.when(c == 0)
        def _():
            pltpu.async_copy(xlo_hbm.at[src_v], rows_v, sem).wait()

        @pl.when(c == 1)
        def _():
            pltpu.async_copy(xhi_hbm.at[src_v], rows_v, sem).wait()

        pltpu.sync_copy(rows_v, aggx_s.at[dst_v], add=True)

        @pl.when((g < EF_SPLIT) == (c == 0))
        def _():
            pltpu.sync_copy(ef_hbm.at[pl.ds(base, CHUNK)], ef_v)
            pltpu.sync_copy(ef_v, agge_s.at[dst_v], add=True)

        return carry

    lax.fori_loop(0, CHUNKS_PER_TILE, body, 0)
    plsc.subcore_barrier()

    out_base = c * N_PAD + r0
    for k in range(ROWS_PER_TILE // CHUNK):
        pltpu.sync_copy(aggx_s.at[pl.ds(r0 + k * CHUNK, CHUNK)], rows_v)
        pltpu.sync_copy(rows_v, outx_hbm.at[pl.ds(out_base + k * CHUNK, CHUNK)])
    pltpu.sync_copy(agge_s.at[pl.ds(r0, ROWS_PER_TILE)], eb_v)
    pltpu.sync_copy(eb_v, oute_hbm.at[pl.ds(out_base, ROWS_PER_TILE)])


BLK = 1280


def _mm_body(axl_ref, axh_ref, ae_ref, wxl_ref, wxh_ref, we_ref, b_ref, o_ref):
    ae = ae_ref[0] + ae_ref[1]
    acc = lax.dot_general(axl_ref[...], wxl_ref[...], (((1,), (0,)), ((), ())),
                          preferred_element_type=jnp.float32,
                          precision=lax.Precision.HIGHEST)
    acc = acc + lax.dot_general(axh_ref[...], wxh_ref[...],
                                (((1,), (0,)), ((), ())),
                                preferred_element_type=jnp.float32,
                                precision=lax.Precision.HIGHEST)
    acc = acc + lax.dot_general(ae, we_ref[...], (((1,), (0,)), ((), ())),
                                preferred_element_type=jnp.float32,
                                precision=lax.Precision.HIGHEST)
    o_ref[...] = acc + b_ref[...]


def kernel(node_feature, edge_index, edge_feature, W, b):
    edge_index = edge_index.astype(jnp.int32)
    src = jnp.ravel(edge_index[0])
    dst = jnp.ravel(edge_index[1])
    xlo = node_feature[:, :D_HALF]
    xhi = node_feature[:, D_HALF:]
    zx = jnp.zeros((CHUNK, D_HALF), jnp.float32)
    ze = jnp.zeros((ROWS_PER_TILE, D_EDGE), jnp.float32)
    outx, oute = _sc_aggregate(src, dst, xlo, xhi, edge_feature, zx, ze)
    axl = outx[:N_PAD]
    axh = outx[N_PAD:]
    ae = oute.reshape(NC, N_PAD, D_EDGE)
    wxl = W[:, :D_HALF].T
    wxh = W[:, D_HALF:D_FEAT].T
    we = W[:, D_FEAT:].T
    out = pl.pallas_call(
        _mm_body,
        grid=(N_PAD // BLK,),
        in_specs=[
            pl.BlockSpec((BLK, D_HALF), lambda i: (i, 0)),
            pl.BlockSpec((BLK, D_HALF), lambda i: (i, 0)),
            pl.BlockSpec((NC, BLK, D_EDGE), lambda i: (0, i, 0)),
            pl.BlockSpec((D_HALF, D_OUT), lambda i: (0, 0)),
            pl.BlockSpec((D_HALF, D_OUT), lambda i: (0, 0)),
            pl.BlockSpec((D_EDGE, D_OUT), lambda i: (0, 0)),
            pl.BlockSpec((1, D_OUT), lambda i: (0, 0)),
        ],
        out_specs=pl.BlockSpec((BLK, D_OUT), lambda i: (i, 0)),
        out_shape=jax.ShapeDtypeStruct((N_PAD, D_OUT), jnp.float32),
    )(axl, axh, ae, wxl, wxh, we, b.reshape(1, D_OUT))
    return out[:N_NODES]


# speedup vs baseline: 4.2073x; 1.4008x over previous
import functools

import jax
import jax.numpy as jnp
from jax import lax
from jax.experimental import pallas as pl
from jax.experimental.pallas import tpu as pltpu
from jax.experimental.pallas import tpu_sc as plsc

N_NODES = 10000
N_PAD = 10240
E_EDGES = 320000
D_FEAT = 128
D_HALF = D_FEAT // 2
D_EDGE = 16
D_OUT = 128

NC = 2
NS = 16
CHUNK = 128
CPT = 158
E_PER_TILE = CPT * CHUNK
E_PAD = NS * E_PER_TILE
EF_SPLIT = CPT // 2
PAIRS = CPT // 2
ROWS_PER_TILE = N_PAD // NS
ZCH = 80

_MESH = plsc.VectorSubcoreMesh(core_axis_name="c", subcore_axis_name="s")


@functools.partial(
    pl.kernel,
    out_type=(
        jax.ShapeDtypeStruct((NC * N_PAD, D_HALF), jnp.float32),
        jax.ShapeDtypeStruct((NC * N_PAD, D_EDGE), jnp.float32),
    ),
    mesh=_MESH,
    compiler_params=pltpu.CompilerParams(use_tc_tiling_on_sc=False),
    scratch_types=[
        pltpu.VMEM((CPT, CHUNK), jnp.int32),
        pltpu.VMEM((CPT, CHUNK), jnp.int32),
        pltpu.VMEM((CHUNK, D_HALF), jnp.float32),
        pltpu.VMEM((CHUNK, D_HALF), jnp.float32),
        pltpu.VMEM((CHUNK, D_EDGE), jnp.float32),
        pltpu.VMEM((ROWS_PER_TILE, D_EDGE), jnp.float32),
        pltpu.VMEM_SHARED((N_PAD, D_HALF), jnp.float32),
        pltpu.VMEM_SHARED((N_PAD, D_EDGE), jnp.float32),
        pltpu.SemaphoreType.DMA,
        pltpu.SemaphoreType.DMA,
    ],
)
def _sc_aggregate(srclo_hbm, srchi_hbm, dst_hbm, xcat_hbm, ef_hbm,
                  zx_hbm, ze_hbm, outx_hbm, oute_hbm,
                  sidx_v, didx_v, rows0_v, rows1_v, ef_v, eb_v,
                  aggx_s, agge_s, semA, semB):
    c = lax.axis_index("c")
    s = lax.axis_index("s")

    @pl.when(c == 0)
    def _():
        pltpu.sync_copy(srclo_hbm.at[pl.ds(s * CPT, CPT)], sidx_v)

    @pl.when(c == 1)
    def _():
        pltpu.sync_copy(srchi_hbm.at[pl.ds(s * CPT, CPT)], sidx_v)

    pltpu.sync_copy(dst_hbm.at[pl.ds(s * CPT, CPT)], didx_v)

    r0 = s * ROWS_PER_TILE
    pltpu.sync_copy(zx_hbm.at[pl.ds(0, ZCH)], rows0_v.at[pl.ds(0, ZCH)])
    pltpu.sync_copy(ze_hbm.at[pl.ds(0, ROWS_PER_TILE)], eb_v)
    for k in range(ROWS_PER_TILE // ZCH):
        pltpu.sync_copy(rows0_v.at[pl.ds(0, ZCH)],
                        aggx_s.at[pl.ds(r0 + k * ZCH, ZCH)])
    pltpu.sync_copy(eb_v, agge_s.at[pl.ds(r0, ROWS_PER_TILE)])
    plsc.subcore_barrier()

    ef_base = s * E_PER_TILE

    def do_ef(g):
        @pl.when((g < EF_SPLIT) == (c == 0))
        def _():
            pltpu.sync_copy(ef_hbm.at[pl.ds(ef_base + g * CHUNK, CHUNK)], ef_v)
            pltpu.sync_copy(ef_v, agge_s.at[didx_v.at[g]], add=True)

    pltpu.async_copy(xcat_hbm.at[sidx_v.at[0]], rows0_v, semA)
    pltpu.async_copy(xcat_hbm.at[sidx_v.at[1]], rows1_v, semB)

    def body(p, carry):
        g0 = 2 * p
        g1 = 2 * p + 1
        pltpu.make_async_copy(xcat_hbm.at[sidx_v.at[g0]], rows0_v, semA).wait()
        pltpu.sync_copy(rows0_v, aggx_s.at[didx_v.at[g0]], add=True)

        @pl.when(g0 + 2 < CPT)
        def _():
            pltpu.async_copy(xcat_hbm.at[sidx_v.at[g0 + 2]], rows0_v, semA)

        do_ef(g0)

        pltpu.make_async_copy(xcat_hbm.at[sidx_v.at[g1]], rows1_v, semB).wait()
        pltpu.sync_copy(rows1_v, aggx_s.at[didx_v.at[g1]], add=True)

        @pl.when(g1 + 2 < CPT)
        def _():
            pltpu.async_copy(xcat_hbm.at[sidx_v.at[g1 + 2]], rows1_v, semB)

        do_ef(g1)
        return carry

    lax.fori_loop(0, PAIRS, body, 0)
    plsc.subcore_barrier()

    out_base = c * N_PAD + r0
    for k in range(ROWS_PER_TILE // CHUNK):
        pltpu.sync_copy(aggx_s.at[pl.ds(r0 + k * CHUNK, CHUNK)], rows0_v)
        pltpu.sync_copy(rows0_v, outx_hbm.at[pl.ds(out_base + k * CHUNK, CHUNK)])
    pltpu.sync_copy(agge_s.at[pl.ds(r0, ROWS_PER_TILE)], eb_v)
    pltpu.sync_copy(eb_v, oute_hbm.at[pl.ds(out_base, ROWS_PER_TILE)])


BLK = 1280


def _mm_body(axl_ref, axh_ref, ae_ref, wxl_ref, wxh_ref, we_ref, b_ref, o_ref):
    ae = ae_ref[0] + ae_ref[1]
    acc = lax.dot_general(axl_ref[...], wxl_ref[...], (((1,), (0,)), ((), ())),
                          preferred_element_type=jnp.float32,
                          precision=lax.Precision.HIGHEST)
    acc = acc + lax.dot_general(axh_ref[...], wxh_ref[...],
                                (((1,), (0,)), ((), ())),
                                preferred_element_type=jnp.float32,
                                precision=lax.Precision.HIGHEST)
    acc = acc + lax.dot_general(ae, we_ref[...], (((1,), (0,)), ((), ())),
                                preferred_element_type=jnp.float32,
                                precision=lax.Precision.HIGHEST)
    o_ref[...] = acc + b_ref[...]


def kernel(node_feature, edge_index, edge_feature, W, b):
    edge_index = edge_index.astype(jnp.int32)
    n_extra = E_PAD - E_EDGES
    src = jnp.concatenate(
        [jnp.ravel(edge_index[0]), jnp.zeros((n_extra,), jnp.int32)])
    dst = jnp.concatenate(
        [jnp.ravel(edge_index[1]),
         jnp.full((n_extra,), N_PAD - 1, jnp.int32)])
    efp = jnp.concatenate(
        [edge_feature, jnp.zeros((n_extra, D_EDGE), jnp.float32)])
    srclo = src.reshape(NS * CPT, CHUNK)
    srchi = (src + N_NODES).reshape(NS * CPT, CHUNK)
    dst2d = dst.reshape(NS * CPT, CHUNK)
    xcat = jnp.concatenate(
        [node_feature[:, :D_HALF], node_feature[:, D_HALF:]], axis=0)
    zx = jnp.zeros((ZCH, D_HALF), jnp.float32)
    ze = jnp.zeros((ROWS_PER_TILE, D_EDGE), jnp.float32)
    outx, oute = _sc_aggregate(srclo, srchi, dst2d, xcat, efp, zx, ze)
    axl = outx[:N_PAD]
    axh = outx[N_PAD:]
    ae = oute.reshape(NC, N_PAD, D_EDGE)
    wxl = W[:, :D_HALF].T
    wxh = W[:, D_HALF:D_FEAT].T
    we = W[:, D_FEAT:].T
    out = pl.pallas_call(
        _mm_body,
        grid=(N_PAD // BLK,),
        in_specs=[
            pl.BlockSpec((BLK, D_HALF), lambda i: (i, 0)),
            pl.BlockSpec((BLK, D_HALF), lambda i: (i, 0)),
            pl.BlockSpec((NC, BLK, D_EDGE), lambda i: (0, i, 0)),
            pl.BlockSpec((D_HALF, D_OUT), lambda i: (0, 0)),
            pl.BlockSpec((D_HALF, D_OUT), lambda i: (0, 0)),
            pl.BlockSpec((D_EDGE, D_OUT), lambda i: (0, 0)),
            pl.BlockSpec((1, D_OUT), lambda i: (0, 0)),
        ],
        out_specs=pl.BlockSpec((BLK, D_OUT), lambda i: (i, 0)),
        out_shape=jax.ShapeDtypeStruct((N_PAD, D_OUT), jnp.float32),
    )(axl, axh, ae, wxl, wxh, we, b.reshape(1, D_OUT))
    return out[:N_NODES]


# speedup vs baseline: 6.3382x; 1.5065x over previous
import functools

import jax
import jax.numpy as jnp
from jax import lax
from jax.experimental import pallas as pl
from jax.experimental.pallas import tpu as pltpu
from jax.experimental.pallas import tpu_sc as plsc

N_NODES = 10000
N_PAD = 10240
E_EDGES = 320000
D_FEAT = 128
D_HALF = D_FEAT // 2
D_EDGE = 16
D_OUT = 128

NC = 2
NS = 16
CHUNK = 80
EPT = E_EDGES // NS
CPT = EPT // CHUNK
NSLOT = 2
ITERS = CPT // NSLOT
EF_ITERS = ITERS // 2
ROWS_PER_TILE = N_PAD // NS

_MESH = plsc.VectorSubcoreMesh(core_axis_name="c", subcore_axis_name="s")


@functools.partial(
    pl.kernel,
    out_type=(
        jax.ShapeDtypeStruct((NC * N_PAD, D_HALF), jnp.float32),
        jax.ShapeDtypeStruct((NC * N_PAD, D_EDGE), jnp.float32),
    ),
    mesh=_MESH,
    compiler_params=pltpu.CompilerParams(use_tc_tiling_on_sc=False),
    scratch_types=[
        pltpu.VMEM((EPT,), jnp.int32),
        pltpu.VMEM((EPT,), jnp.int32),
        [pltpu.VMEM((CHUNK,), jnp.int32)] * NSLOT,
        [pltpu.VMEM((CHUNK, D_HALF), jnp.float32)] * NSLOT,
        [pltpu.VMEM((CHUNK, D_EDGE), jnp.float32)] * NSLOT,
        pltpu.VMEM((ROWS_PER_TILE, D_EDGE), jnp.float32),
        pltpu.VMEM_SHARED((N_PAD, D_HALF), jnp.float32),
        pltpu.VMEM_SHARED((N_PAD, D_EDGE), jnp.float32),
        [pltpu.SemaphoreType.DMA] * NSLOT,
        [pltpu.SemaphoreType.DMA] * NSLOT,
        [pltpu.SemaphoreType.DMA] * NSLOT,
        [pltpu.SemaphoreType.DMA] * NSLOT,
    ],
)
def _sc_aggregate(srclo_hbm, srchi_hbm, dst_hbm, xc_hbm, ef_hbm,
                  zx_hbm, ze_hbm, outx_hbm, oute_hbm,
                  sidx_v, didx_v, dstv, rows, efv, eb_v,
                  aggx_s, agge_s, sem_g, sem_s, sem_el, sem_es):
    c = lax.axis_index("c")
    s = lax.axis_index("s")

    @pl.when(c == 0)
    def _():
        pltpu.sync_copy(srclo_hbm.at[pl.ds(s * EPT, EPT)], sidx_v)

    @pl.when(c == 1)
    def _():
        pltpu.sync_copy(srchi_hbm.at[pl.ds(s * EPT, EPT)], sidx_v)

    pltpu.sync_copy(dst_hbm.at[pl.ds(s * EPT, EPT)], didx_v)

    r0 = s * ROWS_PER_TILE
    pltpu.sync_copy(zx_hbm.at[pl.ds(0, CHUNK)], rows[0])
    pltpu.sync_copy(ze_hbm.at[pl.ds(0, ROWS_PER_TILE)], eb_v)
    for k in range(ROWS_PER_TILE // CHUNK):
        pltpu.sync_copy(rows[0], aggx_s.at[pl.ds(r0 + k * CHUNK, CHUNK)])
    pltpu.sync_copy(eb_v, agge_s.at[pl.ds(r0, ROWS_PER_TILE)])
    plsc.subcore_barrier()

    ef_base = s * EPT

    def gather_wait(j, g):
        pltpu.make_async_copy(
            xc_hbm.at[sidx_v.at[pl.ds(g * CHUNK, CHUNK)]], rows[j],
            sem_g[j]).wait()

    def gather_issue(j, g):
        for t in range(CHUNK // 16):
            dstv[j][pl.ds(16 * t, 16)] = didx_v[pl.ds(g * CHUNK + 16 * t, 16)]
        pltpu.async_copy(
            xc_hbm.at[sidx_v.at[pl.ds(g * CHUNK, CHUNK)]], rows[j], sem_g[j])

    def ef_issue(j, g):
        pltpu.async_copy(ef_hbm.at[pl.ds(ef_base + g * CHUNK, CHUNK)],
                         efv[j], sem_el[j])

    for j in range(NSLOT):
        gather_issue(j, j)

    @pl.when(c == 0)
    def _():
        for j in range(NSLOT):
            ef_issue(j, j)

    def body(i, carry):
        own_ef = (i < EF_ITERS) == (c == 0)
        own_ef_next = ((i + 1) < EF_ITERS) == (c == 0)

        for j in range(NSLOT):
            g = i * NSLOT + j
            gather_wait(j, g)
            pltpu.async_copy(rows[j], aggx_s.at[dstv[j]], sem_s[j], add=True)

            @pl.when(own_ef)
            def _(j=j):
                pltpu.make_async_copy(
                    ef_hbm.at[pl.ds(0, CHUNK)], efv[j], sem_el[j]).wait()
                pltpu.async_copy(efv[j], agge_s.at[dstv[j]], sem_es[j],
                                 add=True)

        @pl.when(i + 1 < ITERS)
        def _():
            for j in range(NSLOT):
                gn = (i + 1) * NSLOT + j
                pltpu.make_async_copy(rows[j], aggx_s.at[dstv[j]],
                                      sem_s[j]).wait()

                @pl.when(own_ef)
                def _(j=j):
                    pltpu.make_async_copy(efv[j], agge_s.at[dstv[j]],
                                          sem_es[j]).wait()

                gather_issue(j, gn)

                @pl.when(own_ef_next)
                def _(j=j, gn=gn):
                    ef_issue(j, gn)

        return carry

    lax.fori_loop(0, ITERS, body, 0)

    for j in range(NSLOT):
        pltpu.make_async_copy(rows[j], aggx_s.at[dstv[j]], sem_s[j]).wait()

        @pl.when(c == 1)
        def _(j=j):
            pltpu.make_async_copy(efv[j], agge_s.at[dstv[j]],
                                  sem_es[j]).wait()

    plsc.subcore_barrier()

    out_base = c * N_PAD + r0
    for k in range(ROWS_PER_TILE // CHUNK):
        pltpu.sync_copy(aggx_s.at[pl.ds(r0 + k * CHUNK, CHUNK)], rows[0])
        pltpu.sync_copy(rows[0], outx_hbm.at[pl.ds(out_base + k * CHUNK,
                                                   CHUNK)])
    pltpu.sync_copy(agge_s.at[pl.ds(r0, ROWS_PER_TILE)], eb_v)
    pltpu.sync_copy(eb_v, oute_hbm.at[pl.ds(out_base, ROWS_PER_TILE)])


BLK = 1280


def _mm_body(axl_ref, axh_ref, ae_ref, wxl_ref, wxh_ref, we_ref, b_ref, o_ref):
    ae = ae_ref[0] + ae_ref[1]
    acc = lax.dot_general(axl_ref[...], wxl_ref[...], (((1,), (0,)), ((), ())),
                          preferred_element_type=jnp.float32,
                          precision=lax.Precision.HIGHEST)
    acc = acc + lax.dot_general(axh_ref[...], wxh_ref[...],
                                (((1,), (0,)), ((), ())),
                                preferred_element_type=jnp.float32,
                                precision=lax.Precision.HIGHEST)
    acc = acc + lax.dot_general(ae, we_ref[...], (((1,), (0,)), ((), ())),
                                preferred_element_type=jnp.float32,
                                precision=lax.Precision.HIGHEST)
    o_ref[...] = acc + b_ref[...]


def kernel(node_feature, edge_index, edge_feature, W, b):
    edge_index = edge_index.astype(jnp.int32)
    src2 = 2 * jnp.ravel(edge_index[0])
    srclo = src2
    srchi = src2 + 1
    dst = jnp.ravel(edge_index[1])
    xc = node_feature.reshape(2 * N_NODES, D_HALF)
    zx = jnp.zeros((CHUNK, D_HALF), jnp.float32)
    ze = jnp.zeros((ROWS_PER_TILE, D_EDGE), jnp.float32)
    outx, oute = _sc_aggregate(srclo, srchi, dst, xc, edge_feature, zx, ze)
    axl = outx[:N_PAD]
    axh = outx[N_PAD:]
    ae = oute.reshape(NC, N_PAD, D_EDGE)
    wxl = W[:, :D_HALF].T
    wxh = W[:, D_HALF:D_FEAT].T
    we = W[:, D_FEAT:].T
    out = pl.pallas_call(
        _mm_body,
        grid=(N_PAD // BLK,),
        in_specs=[
            pl.BlockSpec((BLK, D_HALF), lambda i: (i, 0)),
            pl.BlockSpec((BLK, D_HALF), lambda i: (i, 0)),
            pl.BlockSpec((NC, BLK, D_EDGE), lambda i: (0, i, 0)),
            pl.BlockSpec((D_HALF, D_OUT), lambda i: (0, 0)),
            pl.BlockSpec((D_HALF, D_OUT), lambda i: (0, 0)),
            pl.BlockSpec((D_EDGE, D_OUT), lambda i: (0, 0)),
            pl.BlockSpec((1, D_OUT), lambda i: (0, 0)),
        ],
        out_specs=pl.BlockSpec((BLK, D_OUT), lambda i: (i, 0)),
        out_shape=jax.ShapeDtypeStruct((N_PAD, D_OUT), jnp.float32),
    )(axl, axh, ae, wxl, wxh, we, b.reshape(1, D_OUT))
    return out[:N_NODES]


# speedup vs baseline: 6.4893x; 1.0238x over previous
import functools

import jax
import jax.numpy as jnp
from jax import lax
from jax.experimental import pallas as pl
from jax.experimental.pallas import tpu as pltpu
from jax.experimental.pallas import tpu_sc as plsc

N_NODES = 10000
N_PAD = 10240
E_EDGES = 320000
D_FEAT = 128
D_HALF = D_FEAT // 2
D_EDGE = 16
D_OUT = 128

NC = 2
NS = 16
CHUNK = 80
EPT = E_EDGES // NS
CPT = EPT // CHUNK
NSLOT = 2
ITERS = CPT // NSLOT
EF_ITERS = ITERS // 2
ROWS_PER_TILE = N_PAD // NS

_MESH = plsc.VectorSubcoreMesh(core_axis_name="c", subcore_axis_name="s")


@functools.partial(
    pl.kernel,
    out_type=(
        jax.ShapeDtypeStruct((NC * N_PAD, D_HALF), jnp.float32),
        jax.ShapeDtypeStruct((NC * N_PAD, D_EDGE), jnp.float32),
    ),
    mesh=_MESH,
    compiler_params=pltpu.CompilerParams(use_tc_tiling_on_sc=False),
    scratch_types=[
        pltpu.VMEM((EPT,), jnp.int32),
        pltpu.VMEM((EPT,), jnp.int32),
        [pltpu.VMEM((CHUNK,), jnp.int32)] * NSLOT,
        [pltpu.VMEM((CHUNK, D_HALF), jnp.float32)] * NSLOT,
        [pltpu.VMEM((CHUNK * D_EDGE,), jnp.float32)] * NSLOT,
        [pltpu.VMEM((CHUNK, D_EDGE), jnp.float32)] * NSLOT,
        pltpu.VMEM((ROWS_PER_TILE, D_EDGE), jnp.float32),
        pltpu.VMEM_SHARED((N_PAD, D_HALF), jnp.float32),
        pltpu.VMEM_SHARED((N_PAD, D_EDGE), jnp.float32),
        [pltpu.SemaphoreType.DMA] * NSLOT,
        [pltpu.SemaphoreType.DMA] * NSLOT,
        [pltpu.SemaphoreType.DMA] * NSLOT,
        [pltpu.SemaphoreType.DMA] * NSLOT,
    ],
)
def _sc_aggregate(srclo_hbm, srchi_hbm, dst_hbm, xc_hbm, ef_hbm,
                  zx_hbm, ze_hbm, outx_hbm, oute_hbm,
                  sidx_v, didx_v, dstv, rows, efl, efv, eb_v,
                  aggx_s, agge_s, sem_g, sem_s, sem_el, sem_es):
    c = lax.axis_index("c")
    s = lax.axis_index("s")

    @pl.when(c == 0)
    def _():
        pltpu.sync_copy(srclo_hbm.at[pl.ds(s * EPT, EPT)], sidx_v)

    @pl.when(c == 1)
    def _():
        pltpu.sync_copy(srchi_hbm.at[pl.ds(s * EPT, EPT)], sidx_v)

    pltpu.sync_copy(dst_hbm.at[pl.ds(s * EPT, EPT)], didx_v)

    r0 = s * ROWS_PER_TILE
    pltpu.sync_copy(zx_hbm.at[pl.ds(0, CHUNK)], rows[0])
    pltpu.sync_copy(ze_hbm.at[pl.ds(0, ROWS_PER_TILE)], eb_v)
    for k in range(ROWS_PER_TILE // CHUNK):
        pltpu.sync_copy(rows[0], aggx_s.at[pl.ds(r0 + k * CHUNK, CHUNK)])
    pltpu.sync_copy(eb_v, agge_s.at[pl.ds(r0, ROWS_PER_TILE)])
    plsc.subcore_barrier()

    ef_base = s * EPT

    def gather_wait(j, g):
        pltpu.make_async_copy(
            xc_hbm.at[sidx_v.at[pl.ds(g * CHUNK, CHUNK)]], rows[j],
            sem_g[j]).wait()

    def gather_issue(j, g):
        for t in range(CHUNK // 16):
            dstv[j][pl.ds(16 * t, 16)] = didx_v[pl.ds(g * CHUNK + 16 * t, 16)]
        pltpu.async_copy(
            xc_hbm.at[sidx_v.at[pl.ds(g * CHUNK, CHUNK)]], rows[j], sem_g[j])

    def ef_issue(j, g):
        pltpu.async_copy(
            ef_hbm.at[pl.ds((ef_base + g * CHUNK) * D_EDGE, CHUNK * D_EDGE)],
            efl[j], sem_el[j])

    for j in range(NSLOT):
        gather_issue(j, j)

    @pl.when(c == 0)
    def _():
        for j in range(NSLOT):
            ef_issue(j, j)

    def body(i, carry):
        own_ef = (i < EF_ITERS) == (c == 0)
        own_ef_next = ((i + 1) < EF_ITERS) == (c == 0)

        for j in range(NSLOT):
            g = i * NSLOT + j
            gather_wait(j, g)
            pltpu.async_copy(rows[j], aggx_s.at[dstv[j]], sem_s[j], add=True)

            @pl.when(own_ef)
            def _(j=j):
                pltpu.make_async_copy(
                    ef_hbm.at[pl.ds(0, CHUNK * D_EDGE)], efl[j],
                    sem_el[j]).wait()
                for e in range(CHUNK):
                    efv[j][e, :] = efl[j][pl.ds(e * D_EDGE, D_EDGE)]
                pltpu.async_copy(efv[j], agge_s.at[dstv[j]], sem_es[j],
                                 add=True)

        @pl.when(i + 1 < ITERS)
        def _():
            for j in range(NSLOT):
                gn = (i + 1) * NSLOT + j
                pltpu.make_async_copy(rows[j], aggx_s.at[dstv[j]],
                                      sem_s[j]).wait()

                @pl.when(own_ef)
                def _(j=j):
                    pltpu.make_async_copy(efv[j], agge_s.at[dstv[j]],
                                          sem_es[j]).wait()

                gather_issue(j, gn)

                @pl.when(own_ef_next)
                def _(j=j, gn=gn):
                    ef_issue(j, gn)

        return carry

    lax.fori_loop(0, ITERS, body, 0)

    for j in range(NSLOT):
        pltpu.make_async_copy(rows[j], aggx_s.at[dstv[j]], sem_s[j]).wait()

        @pl.when(c == 1)
        def _(j=j):
            pltpu.make_async_copy(efv[j], agge_s.at[dstv[j]],
                                  sem_es[j]).wait()

    plsc.subcore_barrier()

    out_base = c * N_PAD + r0
    for k in range(ROWS_PER_TILE // CHUNK):
        pltpu.sync_copy(aggx_s.at[pl.ds(r0 + k * CHUNK, CHUNK)], rows[0])
        pltpu.sync_copy(rows[0], outx_hbm.at[pl.ds(out_base + k * CHUNK,
                                                   CHUNK)])
    pltpu.sync_copy(agge_s.at[pl.ds(r0, ROWS_PER_TILE)], eb_v)
    pltpu.sync_copy(eb_v, oute_hbm.at[pl.ds(out_base, ROWS_PER_TILE)])


BLK = 1280


def _mm_body(axl_ref, axh_ref, ae0_ref, ae1_ref, wxl_ref, wxh_ref, we_ref,
             b_ref, o_ref):
    ae = ae0_ref[...] + ae1_ref[...]
    acc = lax.dot_general(axl_ref[...], wxl_ref[...], (((1,), (0,)), ((), ())),
                          preferred_element_type=jnp.float32,
                          precision=lax.Precision.HIGHEST)
    acc = acc + lax.dot_general(axh_ref[...], wxh_ref[...],
                                (((1,), (0,)), ((), ())),
                                preferred_element_type=jnp.float32,
                                precision=lax.Precision.HIGHEST)
    acc = acc + lax.dot_general(ae, we_ref[...], (((1,), (0,)), ((), ())),
                                preferred_element_type=jnp.float32,
                                precision=lax.Precision.HIGHEST)
    o_ref[...] = acc + b_ref[...]


def kernel(node_feature, edge_index, edge_feature, W, b):
    edge_index = edge_index.astype(jnp.int32)
    src2 = 2 * jnp.ravel(edge_index[0])
    srclo = src2
    srchi = src2 + 1
    dst = jnp.ravel(edge_index[1])
    xc = node_feature.reshape(2 * N_NODES, D_HALF)
    zx = jnp.zeros((CHUNK, D_HALF), jnp.float32)
    ze = jnp.zeros((ROWS_PER_TILE, D_EDGE), jnp.float32)
    outx, oute = _sc_aggregate(srclo, srchi, dst, xc,
                               edge_feature.reshape(-1), zx, ze)
    wxl = W[:, :D_HALF].T
    wxh = W[:, D_HALF:D_FEAT].T
    we = W[:, D_FEAT:].T
    out = pl.pallas_call(
        _mm_body,
        grid=(N_PAD // BLK,),
        in_specs=[
            pl.BlockSpec((BLK, D_HALF), lambda i: (i, 0)),
            pl.BlockSpec((BLK, D_HALF), lambda i: (N_PAD // BLK + i, 0)),
            pl.BlockSpec((BLK, D_EDGE), lambda i: (i, 0)),
            pl.BlockSpec((BLK, D_EDGE), lambda i: (N_PAD // BLK + i, 0)),
            pl.BlockSpec((D_HALF, D_OUT), lambda i: (0, 0)),
            pl.BlockSpec((D_HALF, D_OUT), lambda i: (0, 0)),
            pl.BlockSpec((D_EDGE, D_OUT), lambda i: (0, 0)),
            pl.BlockSpec((1, D_OUT), lambda i: (0, 0)),
        ],
        out_specs=pl.BlockSpec((BLK, D_OUT), lambda i: (i, 0)),
        out_shape=jax.ShapeDtypeStruct((N_PAD, D_OUT), jnp.float32),
    )(outx, outx, oute, oute, wxl, wxh, we, b.reshape(1, D_OUT))
    return out[:N_NODES]
